# Initial kernel scaffold; baseline (speedup 1.0000x reference)
import jax
import jax.numpy as jnp
from jax.experimental import pallas as pl


def kernel(context, A_tables, C_last):
    raise NotImplementedError("write your pallas kernel here")


# speedup vs baseline: 27.8578x; 27.8578x over previous
import functools

import jax
import jax.numpy as jnp
from jax import lax
from jax.experimental import pallas as pl
from jax.experimental.pallas import tpu as pltpu
from jax.experimental.pallas import tpu_sc as plsc

B, M, S = 1024, 50, 20
NWORDS, EMB = 100000, 32
N = B * M
NC, NS = 2, 16
NW = NC * NS
SEG_PER_W = N // NW
IDX_PER_W = SEG_PER_W * S
GRP = 128
NGRP = IDX_PER_W // GRP
GPI = 10
NIT = NGRP // GPI
ROWS_PER_IT = GPI * GRP
ZROWS = 160


def _sc_gather_sums(a_resh, c_last, idx_all, dst_all):
  mesh = plsc.VectorSubcoreMesh(core_axis_name="c", subcore_axis_name="s")

  @functools.partial(
      pl.kernel,
      out_type=[jax.ShapeDtypeStruct((N, EMB), jnp.float32)] * 3,
      mesh=mesh,
      compiler_params=pltpu.CompilerParams(use_tc_tiling_on_sc=False),
      scratch_types=[
          pltpu.VMEM((NIT * GPI, GRP), jnp.int32),
          pltpu.VMEM((ROWS_PER_IT,), jnp.int32),
          pltpu.VMEM((ROWS_PER_IT, EMB), jnp.float32),
          pltpu.VMEM((ZROWS, EMB), jnp.float32),
          pltpu.VMEM_SHARED((NS * SEG_PER_W, EMB), jnp.float32),
          pltpu.SemaphoreType.DMA,
      ],
  )
  def k(a_hbm, c_hbm, idx_hbm, dst_hbm, g1_hbm, g2_hbm, gc_hbm,
        dst_v, idx_v, rows_v, zero_v, acc_sh, sem):
    cid = lax.axis_index("c")
    sid = lax.axis_index("s")
    wid = cid * NS + sid

    pltpu.sync_copy(dst_hbm.at[sid], dst_v)

    zvec = jnp.zeros((16,), jnp.float32)

    def zb(i, _):
      zero_v[i, pl.ds(0, 16)] = zvec
      zero_v[i, pl.ds(16, 16)] = zvec
      return 0

    lax.fori_loop(0, ZROWS, zb, 0)

    for t, (src, out) in enumerate(
        ((a_hbm, g1_hbm), (a_hbm, g2_hbm), (c_hbm, gc_hbm))):
      for z in range(SEG_PER_W // ZROWS):
        pltpu.sync_copy(zero_v, acc_sh.at[pl.ds(sid * SEG_PER_W + z * ZROWS,
                                                ZROWS)])

      idx_base = (t * NW + wid) * IDX_PER_W

      def body(it, _, src=src, idx_base=idx_base):
        pltpu.sync_copy(
            idx_hbm.at[pl.ds(idx_base + it * ROWS_PER_IT, ROWS_PER_IT)], idx_v)
        descs = []
        for j in range(GPI):
          descs.append(
              pltpu.async_copy(src.at[idx_v.at[pl.ds(j * GRP, GRP)]],
                               rows_v.at[pl.ds(j * GRP, GRP)], sem))
        for d in descs:
          d.wait()
        for j in range(GPI):
          pltpu.sync_copy(rows_v.at[pl.ds(j * GRP, GRP)],
                          acc_sh.at[dst_v.at[it * GPI + j]], add=True)
        return 0

      lax.fori_loop(0, NIT, body, 0)

      pltpu.sync_copy(acc_sh.at[pl.ds(sid * SEG_PER_W, SEG_PER_W)],
                      out.at[pl.ds(wid * SEG_PER_W, SEG_PER_W)])

  return k(a_resh, c_last, idx_all, dst_all)


def _combine_body(g1_ref, g2_ref, gc_ref, o_ref):
  g1 = g1_ref[...]
  g2 = g2_ref[...]
  gc = gc_ref[...]
  q1 = g1 * (1.0 / EMB)
  t1 = g1 * q1
  e1 = jnp.exp(t1 - jnp.max(t1, axis=-1, keepdims=True))
  a1 = e1 / jnp.sum(e1, axis=-1, keepdims=True)
  q2 = q1 + g2 * a1
  t2 = g2 * q2
  e2 = jnp.exp(t2 - jnp.max(t2, axis=-1, keepdims=True))
  a2 = e2 / jnp.sum(e2, axis=-1, keepdims=True)
  o_ref[...] = gc * a2


def _combine(g1, g2, gc):
  blk = 2048
  spec = pl.BlockSpec((blk, EMB), lambda i: (i, 0))
  return pl.pallas_call(
      _combine_body,
      grid=(N // blk,),
      in_specs=[spec, spec, spec],
      out_specs=spec,
      out_shape=jax.ShapeDtypeStruct((N, EMB), jnp.float32),
  )(g1, g2, gc)


def kernel(context, A_tables, C_last):
  ctx = context.reshape(-1).astype(jnp.int32)
  idx_all = jnp.concatenate([ctx + NWORDS, ctx + 2 * NWORDS, ctx])
  a_resh = A_tables.reshape(3 * NWORDS, EMB)
  r = jax.lax.iota(jnp.int32, IDX_PER_W) // S
  dst_all = (jax.lax.iota(jnp.int32, NS)[:, None] * SEG_PER_W +
             r[None, :]).reshape(NS, NGRP, GRP)
  g1, g2, gc = _sc_gather_sums(a_resh, C_last, idx_all, dst_all)
  out = _combine(g1, g2, gc)
  return out.reshape(B, M, EMB)


# speedup vs baseline: 28.1570x; 1.0107x over previous
import functools

import jax
import jax.numpy as jnp
from jax import lax
from jax.experimental import pallas as pl
from jax.experimental.pallas import tpu as pltpu
from jax.experimental.pallas import tpu_sc as plsc

B, M, S = 1024, 50, 20
NWORDS, EMB = 100000, 32
N = B * M
NC, NS = 2, 16
NW = NC * NS
SEG_PER_W = N // NW
IDX_PER_W = SEG_PER_W * S
GRP = 128
NGRP = IDX_PER_W // GRP
GPI = 10
NIT = NGRP // GPI
ROWS_PER_IT = GPI * GRP
ZROWS = 160


def _sc_gather_sums(a_tabs, c_last, idx_all, dst_all):
  mesh = plsc.VectorSubcoreMesh(core_axis_name="c", subcore_axis_name="s")

  @functools.partial(
      pl.kernel,
      out_type=[jax.ShapeDtypeStruct((N, EMB), jnp.float32)] * 3,
      mesh=mesh,
      compiler_params=pltpu.CompilerParams(use_tc_tiling_on_sc=False),
      scratch_types=[
          pltpu.VMEM((NIT * GPI, GRP), jnp.int32),
          pltpu.VMEM((ROWS_PER_IT,), jnp.int32),
          pltpu.VMEM((ROWS_PER_IT, EMB), jnp.float32),
          pltpu.VMEM((ZROWS, EMB), jnp.float32),
          pltpu.VMEM_SHARED((NS * SEG_PER_W, EMB), jnp.float32),
          pltpu.SemaphoreType.DMA,
      ],
  )
  def k(a_hbm, c_hbm, idx_hbm, dst_hbm, g1_hbm, g2_hbm, gc_hbm,
        dst_v, idx_v, rows_v, zero_v, acc_sh, sem):
    cid = lax.axis_index("c")
    sid = lax.axis_index("s")
    wid = cid * NS + sid

    pltpu.sync_copy(dst_hbm.at[sid], dst_v)

    zvec = jnp.zeros((16,), jnp.float32)

    def zb(i, _):
      zero_v[i, pl.ds(0, 16)] = zvec
      zero_v[i, pl.ds(16, 16)] = zvec
      return 0

    lax.fori_loop(0, ZROWS, zb, 0)

    for t, out in enumerate((g1_hbm, g2_hbm, gc_hbm)):
      src = a_hbm.at[t + 1] if t < 2 else c_hbm
      for z in range(SEG_PER_W // ZROWS):
        pltpu.sync_copy(zero_v, acc_sh.at[pl.ds(sid * SEG_PER_W + z * ZROWS,
                                                ZROWS)])

      idx_base = wid * IDX_PER_W

      def body(it, _, src=src, idx_base=idx_base):
        pltpu.sync_copy(
            idx_hbm.at[pl.ds(idx_base + it * ROWS_PER_IT, ROWS_PER_IT)], idx_v)
        descs = []
        for j in range(GPI):
          descs.append(
              pltpu.async_copy(src.at[idx_v.at[pl.ds(j * GRP, GRP)]],
                               rows_v.at[pl.ds(j * GRP, GRP)], sem))
        for d in descs:
          d.wait()
        for j in range(GPI):
          pltpu.sync_copy(rows_v.at[pl.ds(j * GRP, GRP)],
                          acc_sh.at[dst_v.at[it * GPI + j]], add=True)
        return 0

      lax.fori_loop(0, NIT, body, 0)

      pltpu.sync_copy(acc_sh.at[pl.ds(sid * SEG_PER_W, SEG_PER_W)],
                      out.at[pl.ds(wid * SEG_PER_W, SEG_PER_W)])

  return k(a_tabs, c_last, idx_all, dst_all)


def _combine_body(g1_ref, g2_ref, gc_ref, o_ref):
  g1 = g1_ref[...]
  g2 = g2_ref[...]
  gc = gc_ref[...]
  q1 = g1 * (1.0 / EMB)
  t1 = g1 * q1
  e1 = jnp.exp(t1 - jnp.max(t1, axis=-1, keepdims=True))
  a1 = e1 / jnp.sum(e1, axis=-1, keepdims=True)
  q2 = q1 + g2 * a1
  t2 = g2 * q2
  e2 = jnp.exp(t2 - jnp.max(t2, axis=-1, keepdims=True))
  a2 = e2 / jnp.sum(e2, axis=-1, keepdims=True)
  o_ref[...] = gc * a2


def _combine(g1, g2, gc):
  blk = 2048
  spec = pl.BlockSpec((blk, EMB), lambda i: (i, 0))
  return pl.pallas_call(
      _combine_body,
      grid=(N // blk,),
      in_specs=[spec, spec, spec],
      out_specs=spec,
      out_shape=jax.ShapeDtypeStruct((N, EMB), jnp.float32),
  )(g1, g2, gc)


def kernel(context, A_tables, C_last):
  ctx = context.reshape(-1)
  r = jax.lax.iota(jnp.int32, IDX_PER_W) // S
  dst_all = (jax.lax.iota(jnp.int32, NS)[:, None] * SEG_PER_W +
             r[None, :]).reshape(NS, NGRP, GRP)
  g1, g2, gc = _sc_gather_sums(A_tables, C_last, ctx, dst_all)
  out = _combine(g1, g2, gc)
  return out.reshape(B, M, EMB)


# speedup vs baseline: 30.9173x; 1.0980x over previous
import functools

import jax
import jax.numpy as jnp
from jax import lax
from jax.experimental import pallas as pl
from jax.experimental.pallas import tpu as pltpu
from jax.experimental.pallas import tpu_sc as plsc

B, M, S = 1024, 50, 20
NWORDS, EMB = 100000, 32
N = B * M
NC, NS = 2, 16
NW = NC * NS
SEG_PER_W = N // NW
IDX_PER_W = SEG_PER_W * S
GRP = 128
NGRP = IDX_PER_W // GRP
GPI = 10
NIT = NGRP // GPI
ROWS_PER_IT = GPI * GRP
ZROWS = 160


def _sc_gather_sum(table, ctx, dst_all):
  mesh = plsc.VectorSubcoreMesh(core_axis_name="c", subcore_axis_name="s")

  @functools.partial(
      pl.kernel,
      out_type=jax.ShapeDtypeStruct((N, EMB), jnp.float32),
      mesh=mesh,
      compiler_params=pltpu.CompilerParams(use_tc_tiling_on_sc=False),
      scratch_types=[
          pltpu.VMEM((NGRP, GRP), jnp.int32),
          pltpu.VMEM((ROWS_PER_IT,), jnp.int32),
          pltpu.VMEM((ROWS_PER_IT, EMB), jnp.float32),
          pltpu.VMEM((ZROWS, EMB), jnp.float32),
          pltpu.VMEM_SHARED((NS * SEG_PER_W, EMB), jnp.float32),
          pltpu.SemaphoreType.DMA,
      ],
  )
  def k(t_hbm, idx_hbm, dst_hbm, out_hbm,
        dst_v, idx_v, rows_v, zero_v, acc_sh, sem):
    cid = lax.axis_index("c")
    sid = lax.axis_index("s")
    wid = cid * NS + sid

    pltpu.sync_copy(dst_hbm.at[sid], dst_v)

    zvec = jnp.zeros((16,), jnp.float32)

    def zb(i, _):
      zero_v[i, pl.ds(0, 16)] = zvec
      zero_v[i, pl.ds(16, 16)] = zvec
      return 0

    lax.fori_loop(0, ZROWS, zb, 0)
    for z in range(SEG_PER_W // ZROWS):
      pltpu.sync_copy(zero_v, acc_sh.at[pl.ds(sid * SEG_PER_W + z * ZROWS,
                                              ZROWS)])

    idx_base = wid * IDX_PER_W

    def body(it, _):
      pltpu.sync_copy(
          idx_hbm.at[pl.ds(idx_base + it * ROWS_PER_IT, ROWS_PER_IT)], idx_v)
      descs = []
      for j in range(GPI):
        descs.append(
            pltpu.async_copy(t_hbm.at[idx_v.at[pl.ds(j * GRP, GRP)]],
                             rows_v.at[pl.ds(j * GRP, GRP)], sem))
      for d in descs:
        d.wait()
      for j in range(GPI):
        pltpu.sync_copy(rows_v.at[pl.ds(j * GRP, GRP)],
                        acc_sh.at[dst_v.at[it * GPI + j]], add=True)
      return 0

    lax.fori_loop(0, NIT, body, 0)

    pltpu.sync_copy(acc_sh.at[pl.ds(sid * SEG_PER_W, SEG_PER_W)],
                    out_hbm.at[pl.ds(wid * SEG_PER_W, SEG_PER_W)])

  return k(table, ctx, dst_all)


def _combine_body(g1_ref, g2_ref, gc_ref, o_ref):
  g1 = g1_ref[...]
  g2 = g2_ref[...]
  gc = gc_ref[...]
  q1 = g1 * (1.0 / EMB)
  t1 = g1 * q1
  e1 = jnp.exp(t1 - jnp.max(t1, axis=-1, keepdims=True))
  a1 = e1 / jnp.sum(e1, axis=-1, keepdims=True)
  q2 = q1 + g2 * a1
  t2 = g2 * q2
  e2 = jnp.exp(t2 - jnp.max(t2, axis=-1, keepdims=True))
  a2 = e2 / jnp.sum(e2, axis=-1, keepdims=True)
  o_ref[...] = gc * a2


def _combine(g1, g2, gc):
  blk = 2048
  spec = pl.BlockSpec((blk, EMB), lambda i: (i, 0))
  return pl.pallas_call(
      _combine_body,
      grid=(N // blk,),
      in_specs=[spec, spec, spec],
      out_specs=spec,
      out_shape=jax.ShapeDtypeStruct((N, EMB), jnp.float32),
  )(g1, g2, gc)


def kernel(context, A_tables, C_last):
  ctx = context.reshape(-1)
  r = jax.lax.iota(jnp.int32, IDX_PER_W) // S
  dst_all = (jax.lax.iota(jnp.int32, NS)[:, None] * SEG_PER_W +
             r[None, :]).reshape(NS, NGRP, GRP)
  g1 = _sc_gather_sum(A_tables[1], ctx, dst_all)
  g2 = _sc_gather_sum(A_tables[2], ctx, dst_all)
  gc = _sc_gather_sum(C_last, ctx, dst_all)
  out = _combine(g1, g2, gc)
  return out.reshape(B, M, EMB)


# speedup vs baseline: 33.4644x; 1.0824x over previous
import functools

import jax
import jax.numpy as jnp
from jax import lax
from jax.experimental import pallas as pl
from jax.experimental.pallas import tpu as pltpu
from jax.experimental.pallas import tpu_sc as plsc

B, M, S = 1024, 50, 20
NWORDS, EMB = 100000, 32
N = B * M
NC, NS = 2, 16
NW = NC * NS
SEG_PER_W = N // NW
IDX_PER_W = SEG_PER_W * S
GRP = 128
NGRP = IDX_PER_W // GRP
GPI = 5
NIT = NGRP // GPI
ROWS_PER_IT = GPI * GRP
ZROWS = 160


def _sc_gather_sum(table, ctx, dst_all):
  mesh = plsc.VectorSubcoreMesh(core_axis_name="c", subcore_axis_name="s")

  @functools.partial(
      pl.kernel,
      out_type=jax.ShapeDtypeStruct((N, EMB), jnp.float32),
      mesh=mesh,
      compiler_params=pltpu.CompilerParams(use_tc_tiling_on_sc=False),
      scratch_types=[
          pltpu.VMEM((NGRP, GRP), jnp.int32),
          pltpu.VMEM((ROWS_PER_IT,), jnp.int32),
          pltpu.VMEM((ROWS_PER_IT,), jnp.int32),
          pltpu.VMEM((ROWS_PER_IT, EMB), jnp.float32),
          pltpu.VMEM((ROWS_PER_IT, EMB), jnp.float32),
          pltpu.VMEM((ZROWS, EMB), jnp.float32),
          pltpu.VMEM_SHARED((NS * SEG_PER_W, EMB), jnp.float32),
          pltpu.SemaphoreType.DMA,
      ],
  )
  def k(t_hbm, idx_hbm, dst_hbm, out_hbm,
        dst_v, idx_a, idx_b, rows_a, rows_b, zero_v, acc_sh, sem):
    cid = lax.axis_index("c")
    sid = lax.axis_index("s")
    wid = cid * NS + sid

    pltpu.sync_copy(dst_hbm.at[sid], dst_v)

    zvec = jnp.zeros((16,), jnp.float32)

    def zb(i, _):
      zero_v[i, pl.ds(0, 16)] = zvec
      zero_v[i, pl.ds(16, 16)] = zvec
      return 0

    lax.fori_loop(0, ZROWS, zb, 0)
    for z in range(SEG_PER_W // ZROWS):
      pltpu.sync_copy(zero_v, acc_sh.at[pl.ds(sid * SEG_PER_W + z * ZROWS,
                                              ZROWS)])

    idx_base = wid * IDX_PER_W

    def fire(it, idx_v, rows_v):
      pltpu.sync_copy(
          idx_hbm.at[pl.ds(idx_base + it * ROWS_PER_IT, ROWS_PER_IT)], idx_v)
      for j in range(GPI):
        pltpu.async_copy(t_hbm.at[idx_v.at[pl.ds(j * GRP, GRP)]],
                         rows_v.at[pl.ds(j * GRP, GRP)], sem)

    def drain(rows_v):
      pltpu.make_async_copy(t_hbm.at[idx_a], rows_v, sem).wait()

    def scatter(it, rows_v):
      for j in range(GPI):
        pltpu.sync_copy(rows_v.at[pl.ds(j * GRP, GRP)],
                        acc_sh.at[dst_v.at[it * GPI + j]], add=True)

    fire(0, idx_a, rows_a)

    def body2(k2, _):
      i0 = 2 * k2
      drain(rows_a)
      fire(i0 + 1, idx_b, rows_b)
      scatter(i0, rows_a)
      drain(rows_b)

      @pl.when(i0 + 2 < NIT)
      def _():
        fire(i0 + 2, idx_a, rows_a)

      scatter(i0 + 1, rows_b)
      return 0

    lax.fori_loop(0, NIT // 2, body2, 0)

    pltpu.sync_copy(acc_sh.at[pl.ds(sid * SEG_PER_W, SEG_PER_W)],
                    out_hbm.at[pl.ds(wid * SEG_PER_W, SEG_PER_W)])

  return k(table, ctx, dst_all)


def _combine_body(g1_ref, g2_ref, gc_ref, o_ref):
  g1 = g1_ref[...]
  g2 = g2_ref[...]
  gc = gc_ref[...]
  q1 = g1 * (1.0 / EMB)
  t1 = g1 * q1
  e1 = jnp.exp(t1 - jnp.max(t1, axis=-1, keepdims=True))
  a1 = e1 / jnp.sum(e1, axis=-1, keepdims=True)
  q2 = q1 + g2 * a1
  t2 = g2 * q2
  e2 = jnp.exp(t2 - jnp.max(t2, axis=-1, keepdims=True))
  a2 = e2 / jnp.sum(e2, axis=-1, keepdims=True)
  o_ref[...] = gc * a2


def _combine(g1, g2, gc):
  blk = 2048
  spec = pl.BlockSpec((blk, EMB), lambda i: (i, 0))
  return pl.pallas_call(
      _combine_body,
      grid=(N // blk,),
      in_specs=[spec, spec, spec],
      out_specs=spec,
      out_shape=jax.ShapeDtypeStruct((N, EMB), jnp.float32),
  )(g1, g2, gc)


def kernel(context, A_tables, C_last):
  ctx = context.reshape(-1)
  r = jax.lax.iota(jnp.int32, IDX_PER_W) // S
  dst_all = (jax.lax.iota(jnp.int32, NS)[:, None] * SEG_PER_W +
             r[None, :]).reshape(NS, NGRP, GRP)
  g1 = _sc_gather_sum(A_tables[1], ctx, dst_all)
  g2 = _sc_gather_sum(A_tables[2], ctx, dst_all)
  gc = _sc_gather_sum(C_last, ctx, dst_all)
  out = _combine(g1, g2, gc)
  return out.reshape(B, M, EMB)


# speedup vs baseline: 35.5139x; 1.0612x over previous
import functools

import jax
import jax.numpy as jnp
from jax import lax
from jax.experimental import pallas as pl
from jax.experimental.pallas import tpu as pltpu
from jax.experimental.pallas import tpu_sc as plsc

B, M, S = 1024, 50, 20
NWORDS, EMB = 100000, 32
N = B * M
NC, NS = 2, 16
NW = NC * NS
SEG_PER_W = N // NW
IDX_PER_W = SEG_PER_W * S
GRP = 128
NGRP = IDX_PER_W // GRP
GPI = 5
NIT = NGRP // GPI
ROWS_PER_IT = GPI * GRP
ZROWS = 160


def _sc_gather_sum(table, ctx, dst_all):
  mesh = plsc.VectorSubcoreMesh(core_axis_name="c", subcore_axis_name="s")

  @functools.partial(
      pl.kernel,
      out_type=jax.ShapeDtypeStruct((N, EMB), jnp.float32),
      mesh=mesh,
      compiler_params=pltpu.CompilerParams(use_tc_tiling_on_sc=False),
      scratch_types=[
          pltpu.VMEM((NGRP, GRP), jnp.int32),
          pltpu.VMEM((ROWS_PER_IT,), jnp.int32),
          pltpu.VMEM((ROWS_PER_IT,), jnp.int32),
          pltpu.VMEM((ROWS_PER_IT, EMB), jnp.float32),
          pltpu.VMEM((ROWS_PER_IT, EMB), jnp.float32),
          pltpu.VMEM((ZROWS, EMB), jnp.float32),
          pltpu.VMEM_SHARED((NS * SEG_PER_W, EMB), jnp.float32),
          pltpu.SemaphoreType.DMA,
      ],
  )
  def k(t_hbm, idx_hbm, dst_hbm, out_hbm,
        dst_v, idx_a, idx_b, rows_a, rows_b, zero_v, acc_sh, sem):
    cid = lax.axis_index("c")
    sid = lax.axis_index("s")
    wid = cid * NS + sid

    pltpu.sync_copy(dst_hbm.at[sid], dst_v)

    zvec = jnp.zeros((16,), jnp.float32)

    def zb(i, _):
      zero_v[i, pl.ds(0, 16)] = zvec
      zero_v[i, pl.ds(16, 16)] = zvec
      return 0

    lax.fori_loop(0, ZROWS, zb, 0)
    for z in range(SEG_PER_W // ZROWS):
      pltpu.sync_copy(zero_v, acc_sh.at[pl.ds(sid * SEG_PER_W + z * ZROWS,
                                              ZROWS)])

    idx_base = wid * IDX_PER_W

    def fire(it, idx_v, rows_v):
      pltpu.sync_copy(
          idx_hbm.at[pl.ds(idx_base + it * ROWS_PER_IT, ROWS_PER_IT)], idx_v)
      for j in range(GPI):
        pltpu.async_copy(t_hbm.at[idx_v.at[pl.ds(j * GRP, GRP)]],
                         rows_v.at[pl.ds(j * GRP, GRP)], sem)

    def drain(rows_v):
      pltpu.make_async_copy(t_hbm.at[idx_a], rows_v, sem).wait()

    def scatter(it, rows_v):
      for j in range(GPI):
        pltpu.sync_copy(rows_v.at[pl.ds(j * GRP, GRP)],
                        acc_sh.at[dst_v.at[it * GPI + j]], add=True)

    fire(0, idx_a, rows_a)

    def body2(k2, _):
      i0 = 2 * k2
      drain(rows_a)
      fire(i0 + 1, idx_b, rows_b)
      scatter(i0, rows_a)
      drain(rows_b)

      @pl.when(i0 + 2 < NIT)
      def _():
        fire(i0 + 2, idx_a, rows_a)

      scatter(i0 + 1, rows_b)
      return 0

    lax.fori_loop(0, NIT // 2, body2, 0)

    pltpu.sync_copy(acc_sh.at[pl.ds(sid * SEG_PER_W, SEG_PER_W)],
                    out_hbm.at[pl.ds(wid * SEG_PER_W, SEG_PER_W)])

  return k(table, ctx, dst_all)


def _combine_body(g1_ref, g2_ref, gc_ref, o_ref):
  g1 = g1_ref[...]
  g2 = g2_ref[...]
  gc = gc_ref[...]
  q1 = g1 * (1.0 / EMB)
  t1 = g1 * q1
  e1 = jnp.exp(t1 - jnp.max(t1, axis=-1, keepdims=True))
  a1 = e1 / jnp.sum(e1, axis=-1, keepdims=True)
  q2 = q1 + g2 * a1
  t2 = g2 * q2
  e2 = jnp.exp(t2 - jnp.max(t2, axis=-1, keepdims=True))
  a2 = e2 / jnp.sum(e2, axis=-1, keepdims=True)
  o_ref[...] = gc * a2


def _combine_body_3d(g1_ref, g2_ref, gc_ref, o_ref):
  g1 = g1_ref[...]
  g2 = g2_ref[...]
  gc = gc_ref[...]
  q1 = g1 * (1.0 / EMB)
  t1 = g1 * q1
  e1 = jnp.exp(t1 - jnp.max(t1, axis=-1, keepdims=True))
  a1 = e1 / jnp.sum(e1, axis=-1, keepdims=True)
  q2 = q1 + g2 * a1
  t2 = g2 * q2
  e2 = jnp.exp(t2 - jnp.max(t2, axis=-1, keepdims=True))
  a2 = e2 / jnp.sum(e2, axis=-1, keepdims=True)
  o_ref[...] = (gc * a2).reshape(o_ref.shape)


def _combine(g1, g2, gc):
  bb = 64
  blk = bb * M
  spec = pl.BlockSpec((blk, EMB), lambda i: (i, 0))
  return pl.pallas_call(
      _combine_body_3d,
      grid=(N // blk,),
      in_specs=[spec, spec, spec],
      out_specs=pl.BlockSpec((bb, M, EMB), lambda i: (i, 0, 0)),
      out_shape=jax.ShapeDtypeStruct((B, M, EMB), jnp.float32),
  )(g1, g2, gc)


def kernel(context, A_tables, C_last):
  ctx = context.reshape(-1)
  r = jax.lax.iota(jnp.int32, IDX_PER_W) // S
  dst_all = (jax.lax.iota(jnp.int32, NS)[:, None] * SEG_PER_W +
             r[None, :]).reshape(NS, NGRP, GRP)
  gc = _sc_gather_sum(C_last, ctx, dst_all)
  g1 = _sc_gather_sum(A_tables[1], ctx, dst_all)
  g2 = _sc_gather_sum(A_tables[2], ctx, dst_all)
  return _combine(g1, g2, gc)


# speedup vs baseline: 37.1078x; 1.0449x over previous
import functools

import jax
import jax.numpy as jnp
from jax import lax
from jax.experimental import pallas as pl
from jax.experimental.pallas import tpu as pltpu
from jax.experimental.pallas import tpu_sc as plsc

B, M, S = 1024, 50, 20
NWORDS, EMB = 100000, 32
N = B * M
NC, NS = 2, 16
NW = NC * NS
SEG_PER_W = N // NW
IDX_PER_W = SEG_PER_W * S
GRP = 128
NGRP = IDX_PER_W // GRP
GPI = 5
NIT = NGRP // GPI
ROWS_PER_IT = GPI * GRP
ZROWS = 160


def _sc_gather_sum(table, ctx, dst_all):
  mesh = plsc.VectorSubcoreMesh(core_axis_name="c", subcore_axis_name="s")

  @functools.partial(
      pl.kernel,
      out_type=jax.ShapeDtypeStruct((N, EMB), jnp.float32),
      mesh=mesh,
      compiler_params=pltpu.CompilerParams(use_tc_tiling_on_sc=False),
      scratch_types=[
          pltpu.VMEM((NGRP, GRP), jnp.int32),
          pltpu.VMEM((ROWS_PER_IT,), jnp.int32),
          pltpu.VMEM((ROWS_PER_IT,), jnp.int32),
          pltpu.VMEM((ROWS_PER_IT, EMB), jnp.float32),
          pltpu.VMEM((ROWS_PER_IT, EMB), jnp.float32),
          pltpu.VMEM((ZROWS, EMB), jnp.float32),
          pltpu.VMEM_SHARED((NS * SEG_PER_W, EMB), jnp.float32),
          pltpu.SemaphoreType.DMA,
      ],
  )
  def k(t_hbm, idx_hbm, dst_hbm, out_hbm,
        dst_v, idx_a, idx_b, rows_a, rows_b, zero_v, acc_sh, sem):
    cid = lax.axis_index("c")
    sid = lax.axis_index("s")
    wid = cid * NS + sid

    pltpu.sync_copy(dst_hbm.at[sid], dst_v)

    zvec = jnp.zeros((16,), jnp.float32)

    def zb(i, _):
      zero_v[i, pl.ds(0, 16)] = zvec
      zero_v[i, pl.ds(16, 16)] = zvec
      return 0

    lax.fori_loop(0, ZROWS, zb, 0)
    for z in range(SEG_PER_W // ZROWS):
      pltpu.sync_copy(zero_v, acc_sh.at[pl.ds(sid * SEG_PER_W + z * ZROWS,
                                              ZROWS)])

    idx_base = wid * IDX_PER_W

    def fire(it, idx_v, rows_v):
      pltpu.sync_copy(
          idx_hbm.at[pl.ds(idx_base + it * ROWS_PER_IT, ROWS_PER_IT)], idx_v)
      for j in range(GPI):
        pltpu.async_copy(t_hbm.at[idx_v.at[pl.ds(j * GRP, GRP)]],
                         rows_v.at[pl.ds(j * GRP, GRP)], sem)

    def drain(rows_v):
      pltpu.make_async_copy(t_hbm.at[idx_a], rows_v, sem).wait()

    def scatter(it, rows_v):
      for j in range(GPI):
        pltpu.sync_copy(rows_v.at[pl.ds(j * GRP, GRP)],
                        acc_sh.at[dst_v.at[it * GPI + j]], add=True)

    fire(0, idx_a, rows_a)

    def body2(k2, _):
      i0 = 2 * k2
      drain(rows_a)
      fire(i0 + 1, idx_b, rows_b)
      scatter(i0, rows_a)
      drain(rows_b)

      @pl.when(i0 + 2 < NIT)
      def _():
        fire(i0 + 2, idx_a, rows_a)

      scatter(i0 + 1, rows_b)
      return 0

    lax.fori_loop(0, NIT // 2, body2, 0)

    pltpu.sync_copy(acc_sh.at[pl.ds(sid * SEG_PER_W, SEG_PER_W)],
                    out_hbm.at[pl.ds(wid * SEG_PER_W, SEG_PER_W)])

  return k(table, ctx, dst_all)


def _combine_body(g1_ref, g2_ref, gc_ref, o_ref):
  g1 = g1_ref[...]
  g2 = g2_ref[...]
  gc = gc_ref[...]
  q1 = g1 * (1.0 / EMB)
  t1 = g1 * q1
  e1 = jnp.exp(t1 - jnp.max(t1, axis=-1, keepdims=True))
  a1 = e1 / jnp.sum(e1, axis=-1, keepdims=True)
  q2 = q1 + g2 * a1
  t2 = g2 * q2
  e2 = jnp.exp(t2 - jnp.max(t2, axis=-1, keepdims=True))
  a2 = e2 / jnp.sum(e2, axis=-1, keepdims=True)
  o_ref[...] = gc * a2


def _combine_body_3d(g1_ref, g2_ref, gc_ref, o_ref):
  g1 = g1_ref[...]
  g2 = g2_ref[...]
  gc = gc_ref[...]
  q1 = g1 * (1.0 / EMB)
  t1 = g1 * q1
  e1 = jnp.exp(t1 - jnp.max(t1, axis=-1, keepdims=True))
  a1 = e1 / jnp.sum(e1, axis=-1, keepdims=True)
  q2 = q1 + g2 * a1
  t2 = g2 * q2
  e2 = jnp.exp(t2 - jnp.max(t2, axis=-1, keepdims=True))
  a2 = e2 / jnp.sum(e2, axis=-1, keepdims=True)
  o_ref[...] = (gc * a2).reshape(o_ref.shape)


def _combine(g1, g2, gc):
  bb = 64
  blk = bb * M
  spec = pl.BlockSpec((blk, EMB), lambda i: (i, 0))
  return pl.pallas_call(
      _combine_body_3d,
      grid=(N // blk,),
      in_specs=[spec, spec, spec],
      out_specs=pl.BlockSpec((bb, M, EMB), lambda i: (i, 0, 0)),
      out_shape=jax.ShapeDtypeStruct((B, M, EMB), jnp.float32),
  )(g1, g2, gc)


def kernel(context, A_tables, C_last):
  ctx = context.reshape(NW, SEG_PER_W, S).transpose(0, 2, 1).reshape(-1)
  r = jax.lax.iota(jnp.int32, IDX_PER_W) % SEG_PER_W
  dst_all = (jax.lax.iota(jnp.int32, NS)[:, None] * SEG_PER_W +
             r[None, :]).reshape(NS, NGRP, GRP)
  gc = _sc_gather_sum(C_last, ctx, dst_all)
  g1 = _sc_gather_sum(A_tables[1], ctx, dst_all)
  g2 = _sc_gather_sum(A_tables[2], ctx, dst_all)
  return _combine(g1, g2, gc)


# speedup vs baseline: 38.1286x; 1.0275x over previous
import functools

import jax
import jax.numpy as jnp
from jax import lax
from jax.experimental import pallas as pl
from jax.experimental.pallas import tpu as pltpu
from jax.experimental.pallas import tpu_sc as plsc

B, M, S = 1024, 50, 20
NWORDS, EMB = 100000, 32
N = B * M
NC, NS = 2, 16
NW = NC * NS
SEG_PER_W = N // NW
IDX_PER_W = SEG_PER_W * S
GRP = 128
NGRP = IDX_PER_W // GRP
GPI = 5
NIT = NGRP // GPI
ROWS_PER_IT = GPI * GRP
DSTP = 50
ZROWS = 160


def _sc_gather_sum(table, ctx, dst_all):
  mesh = plsc.VectorSubcoreMesh(core_axis_name="c", subcore_axis_name="s")

  @functools.partial(
      pl.kernel,
      out_type=jax.ShapeDtypeStruct((N, EMB), jnp.float32),
      mesh=mesh,
      compiler_params=pltpu.CompilerParams(use_tc_tiling_on_sc=False),
      scratch_types=[
          pltpu.VMEM((DSTP, GRP), jnp.int32),
          pltpu.VMEM((ROWS_PER_IT,), jnp.int32),
          pltpu.VMEM((ROWS_PER_IT,), jnp.int32),
          pltpu.VMEM((ROWS_PER_IT, EMB), jnp.float32),
          pltpu.VMEM((ROWS_PER_IT, EMB), jnp.float32),
          pltpu.VMEM((ZROWS, EMB), jnp.float32),
          pltpu.VMEM_SHARED((NS * SEG_PER_W, EMB), jnp.float32),
          pltpu.SemaphoreType.DMA,
          pltpu.SemaphoreType.DMA,
      ],
  )
  def k(t_hbm, idx_hbm, dst_hbm, out_hbm,
        dst_v, idx_a, idx_b, rows_a, rows_b, zero_v, acc_sh, gsem, ssem):
    cid = lax.axis_index("c")
    sid = lax.axis_index("s")
    wid = cid * NS + sid

    pltpu.sync_copy(dst_hbm.at[sid], dst_v)

    idx_base = wid * IDX_PER_W

    def fire(it, idx_v, rows_v):
      pltpu.sync_copy(
          idx_hbm.at[pl.ds(idx_base + it * ROWS_PER_IT, ROWS_PER_IT)], idx_v)
      for j in range(GPI):
        pltpu.async_copy(t_hbm.at[idx_v.at[pl.ds(j * GRP, GRP)]],
                         rows_v.at[pl.ds(j * GRP, GRP)], gsem)

    def drain_g(rows_v):
      pltpu.make_async_copy(t_hbm.at[idx_a], rows_v, gsem).wait()

    def scatter(it, rows_v):
      for j in range(GPI):
        pltpu.async_copy(rows_v.at[pl.ds(j * GRP, GRP)],
                         acc_sh.at[dst_v.at[lax.rem(it * GPI + j, DSTP)]],
                         ssem, add=True)

    def drain_s():
      pltpu.make_async_copy(
          t_hbm.at[pl.ds(0, 2 * ROWS_PER_IT)],
          acc_sh.at[pl.ds(sid * SEG_PER_W, 2 * ROWS_PER_IT)], ssem).wait()

    fire(0, idx_a, rows_a)
    zvec = jnp.zeros((16,), jnp.float32)

    def zb(i, _):
      zero_v[i, pl.ds(0, 16)] = zvec
      zero_v[i, pl.ds(16, 16)] = zvec
      return 0

    lax.fori_loop(0, ZROWS, zb, 0)
    for z in range(SEG_PER_W // ZROWS):
      pltpu.sync_copy(zero_v, acc_sh.at[pl.ds(sid * SEG_PER_W + z * ZROWS,
                                              ZROWS)])

    def body2(k2, _):
      i0 = 2 * k2
      drain_g(rows_a)
      fire(i0 + 1, idx_b, rows_b)
      scatter(i0, rows_a)
      drain_g(rows_b)
      scatter(i0 + 1, rows_b)
      drain_s()

      @pl.when(i0 + 2 < NIT)
      def _():
        fire(i0 + 2, idx_a, rows_a)

      return 0

    lax.fori_loop(0, NIT // 2, body2, 0)

    pltpu.sync_copy(acc_sh.at[pl.ds(sid * SEG_PER_W, SEG_PER_W)],
                    out_hbm.at[pl.ds(wid * SEG_PER_W, SEG_PER_W)])

  return k(table, ctx, dst_all)


def _combine_body(g1_ref, g2_ref, gc_ref, o_ref):
  g1 = g1_ref[...]
  g2 = g2_ref[...]
  gc = gc_ref[...]
  q1 = g1 * (1.0 / EMB)
  t1 = g1 * q1
  e1 = jnp.exp(t1 - jnp.max(t1, axis=-1, keepdims=True))
  a1 = e1 / jnp.sum(e1, axis=-1, keepdims=True)
  q2 = q1 + g2 * a1
  t2 = g2 * q2
  e2 = jnp.exp(t2 - jnp.max(t2, axis=-1, keepdims=True))
  a2 = e2 / jnp.sum(e2, axis=-1, keepdims=True)
  o_ref[...] = gc * a2


def _combine_body_3d(g1_ref, g2_ref, gc_ref, o_ref):
  g1 = g1_ref[...]
  g2 = g2_ref[...]
  gc = gc_ref[...]
  q1 = g1 * (1.0 / EMB)
  t1 = g1 * q1
  e1 = jnp.exp(t1 - jnp.max(t1, axis=-1, keepdims=True))
  a1 = e1 / jnp.sum(e1, axis=-1, keepdims=True)
  q2 = q1 + g2 * a1
  t2 = g2 * q2
  e2 = jnp.exp(t2 - jnp.max(t2, axis=-1, keepdims=True))
  a2 = e2 / jnp.sum(e2, axis=-1, keepdims=True)
  o_ref[...] = (gc * a2).reshape(o_ref.shape)


def _combine(g1, g2, gc):
  bb = 64
  blk = bb * M
  spec = pl.BlockSpec((blk, EMB), lambda i: (i, 0))
  return pl.pallas_call(
      _combine_body_3d,
      grid=(N // blk,),
      in_specs=[spec, spec, spec],
      out_specs=pl.BlockSpec((bb, M, EMB), lambda i: (i, 0, 0)),
      out_shape=jax.ShapeDtypeStruct((B, M, EMB), jnp.float32),
  )(g1, g2, gc)


def kernel(context, A_tables, C_last):
  ctx = context.reshape(NW, SEG_PER_W, S).transpose(0, 2, 1).reshape(-1)
  r = jax.lax.iota(jnp.int32, DSTP * GRP) % SEG_PER_W
  dst_all = (jax.lax.iota(jnp.int32, NS)[:, None] * SEG_PER_W +
             r[None, :]).reshape(NS, DSTP, GRP)
  gc = _sc_gather_sum(C_last, ctx, dst_all)
  g1 = _sc_gather_sum(A_tables[1], ctx, dst_all)
  g2 = _sc_gather_sum(A_tables[2], ctx, dst_all)
  return _combine(g1, g2, gc)
